# Initial kernel scaffold; baseline (speedup 1.0000x reference)
import jax
import jax.numpy as jnp
from jax.experimental import pallas as pl


def kernel(hidden, memory, W1, b1, ln_g, ln_b, W2, b2, q_origin, Wp, bp):
    raise NotImplementedError("write your pallas kernel here")


# speedup vs baseline: 5.0492x; 5.0492x over previous
import jax
import jax.numpy as jnp
from jax import lax
from jax.experimental import pallas as pl
from jax.experimental.pallas import tpu as pltpu
from jax.experimental.pallas import tpu_sc as plsc

HID = 1024
MD = 64
M = 100000
Q = 1024
K = 16
ALPHA = 0.1
MAXN = 1.0 - 1e-5

CHUNK = 2048
NCH = 49
MPAD = NCH * CHUNK
TW = 128
BIN = 8
BPC = CHUNK // BIN
NBIN = NCH * BPC
CAND = K * BIN

_Q3 = 128
_Q5 = 64


def _qnet_body(hidden_ref, w1_ref, b1_ref, g_ref, b_ref, w2_ref, b2_ref,
               qo_ref, qp_ref):
    x = jnp.dot(hidden_ref[...], w1_ref[...],
                preferred_element_type=jnp.float32) + b1_ref[...]
    mu = jnp.mean(x, axis=-1, keepdims=True)
    var = jnp.var(x, axis=-1, keepdims=True)
    x = (x - mu) / jnp.sqrt(var + 1e-5) * g_ref[...] + b_ref[...]
    x = jax.nn.gelu(x)
    q = jnp.dot(x, w2_ref[...],
                preferred_element_type=jnp.float32) + b2_ref[...] + qo_ref[...]
    n = jnp.sqrt(jnp.sum(q * q, axis=-1, keepdims=True))
    scale = jnp.minimum(1.0, MAXN / jnp.maximum(n, 1e-12))
    qp_ref[...] = q * scale


def _scan_body(qp_ref, t_ref, a_ref, binmin_ref):
    mem = t_ref[:, :MD]
    n = t_ref[:, MD:MD + 1]
    scale = jnp.minimum(1.0, MAXN / jnp.maximum(n, 1e-12))
    mp = mem * scale
    sm = a_ref[0, 1:2, :]
    invm = 1.0 / jnp.maximum(1.0 - sm, 1e-9)
    qp = qp_ref[...]
    sq = jnp.sum(qp * qp, axis=-1, keepdims=True)
    g = lax.dot_general(qp, mp, (((1,), (1,)), ((), ())),
                        preferred_element_type=jnp.float32)
    r = ((sq + sm) - 2.0 * g) * invm
    binmin_ref[...] = jnp.min(r.reshape(Q, BIN, BPC), axis=1)


def _select_body(binmin_ref, idxr_ref):
    v = binmin_ref[...]
    bid = lax.broadcasted_iota(jnp.int32, (_Q3, NBIN), 1)
    tix = lax.broadcasted_iota(jnp.int32, (_Q3, BIN), 1)
    imax = jnp.int32(2**31 - 1)
    for j in range(K):
        m = jnp.min(v, axis=-1, keepdims=True)
        elig = v <= m
        sel = jnp.min(jnp.where(elig, bid, imax), axis=-1, keepdims=True)
        v = jnp.where(bid == sel, jnp.inf, v)
        base = (sel // BPC) * CHUNK + (sel % BPC)
        idxr_ref[:, j * BIN:(j + 1) * BIN] = base + BPC * tix


def _final_body(qp_ref, rows_ref, idxr_ref, wp_ref, bp_ref, out_ref):
    qp = qp_ref[...]
    raw = rows_ref[:, :, :MD]
    n = rows_ref[:, :, MD]
    sm = rows_ref[:, :, MD + 1]
    idx = idxr_ref[...]
    scale = jnp.minimum(1.0, MAXN / jnp.maximum(n, 1e-12))
    mp = raw * scale[..., None]
    sq = jnp.sum(qp * qp, axis=-1, keepdims=True)
    dot = jnp.sum(mp * qp[:, None, :], axis=-1)
    d2 = jnp.maximum(sq + sm - 2.0 * dot, 0.0)
    denom = jnp.maximum((1.0 - sq) * (1.0 - sm), 1e-12)
    arg = jnp.maximum(1.0 + 2.0 * d2 / denom, 1.0 + 1e-7)
    dist = jnp.log(arg + jnp.sqrt(arg * arg - 1.0))
    dist = jnp.where(idx < M, dist, jnp.inf)
    d0 = dist
    selmask = jnp.zeros(dist.shape, dtype=jnp.bool_)
    imax = jnp.int32(2**31 - 1)
    for j in range(K):
        m = jnp.min(dist, axis=-1, keepdims=True)
        elig = dist <= m
        sel = jnp.min(jnp.where(elig, idx, imax), axis=-1, keepdims=True)
        pick = elig & (idx == sel)
        selmask = selmask | pick
        dist = jnp.where(pick, jnp.inf, dist)
    dmin = jnp.min(d0, axis=-1, keepdims=True)
    w = jnp.where(selmask, jnp.exp((dmin - d0) / ALPHA), 0.0)
    w = w / jnp.sum(w, axis=-1, keepdims=True)
    retrieved = jnp.sum(raw * w[..., None], axis=1)
    out_ref[...] = jnp.dot(retrieved, wp_ref[...],
                           preferred_element_type=jnp.float32) + bp_ref[...]


_SC_B = Q * CAND
_NW = 32
_BPW = _SC_B // _NW
_SC_CH = 512


def _gather_body(tab_hbm, idx_hbm, out_hbm, idx_v, rows_v, sem):
    wid = lax.axis_index("s") * 2 + lax.axis_index("c")
    base = wid * _BPW

    def step(i, carry):
        off = base + i * _SC_CH
        pltpu.sync_copy(idx_hbm.at[pl.ds(off, _SC_CH)], idx_v)
        pltpu.async_copy(tab_hbm.at[idx_v], rows_v, sem).wait()
        pltpu.sync_copy(rows_v, out_hbm.at[pl.ds(off, _SC_CH)])
        return carry

    lax.fori_loop(0, _BPW // _SC_CH, step, 0)


def _gather_rows(table, idx_flat):
    mesh = plsc.VectorSubcoreMesh(core_axis_name="c", subcore_axis_name="s")
    run = pl.kernel(
        _gather_body,
        mesh=mesh,
        out_type=jax.ShapeDtypeStruct((_SC_B, TW), jnp.float32),
        scratch_types=[
            pltpu.VMEM((_SC_CH,), jnp.int32),
            pltpu.VMEM((_SC_CH, TW), jnp.float32),
            pltpu.SemaphoreType.DMA,
        ],
    )
    return run(table, idx_flat)


def _qnet(hidden, W1, b1, ln_g, ln_b, W2, b2, q_origin):
    return pl.pallas_call(
        _qnet_body,
        out_shape=jax.ShapeDtypeStruct((Q, MD), jnp.float32),
    )(hidden, W1, b1.reshape(1, MD), ln_g.reshape(1, MD), ln_b.reshape(1, MD),
      W2, b2.reshape(1, MD), q_origin)


def _scan(qp, table, aux):
    return pl.pallas_call(
        _scan_body,
        grid=(NCH,),
        in_specs=[
            pl.BlockSpec((Q, MD), lambda c: (0, 0)),
            pl.BlockSpec((CHUNK, TW), lambda c: (c, 0)),
            pl.BlockSpec((1, 2, CHUNK), lambda c: (c, 0, 0)),
        ],
        out_specs=pl.BlockSpec((Q, BPC), lambda c: (0, c)),
        out_shape=jax.ShapeDtypeStruct((Q, NBIN), jnp.float32),
    )(qp, table, aux)


def _select(binmin):
    return pl.pallas_call(
        _select_body,
        grid=(Q // _Q3,),
        in_specs=[pl.BlockSpec((_Q3, NBIN), lambda t: (t, 0))],
        out_specs=pl.BlockSpec((_Q3, CAND), lambda t: (t, 0)),
        out_shape=jax.ShapeDtypeStruct((Q, CAND), jnp.int32),
    )(binmin)


def _final(qp, rows, idxr, Wp, bp):
    return pl.pallas_call(
        _final_body,
        grid=(Q // _Q5,),
        in_specs=[
            pl.BlockSpec((_Q5, MD), lambda t: (t, 0)),
            pl.BlockSpec((_Q5, CAND, TW), lambda t: (t, 0, 0)),
            pl.BlockSpec((_Q5, CAND), lambda t: (t, 0)),
            pl.BlockSpec((MD, HID), lambda t: (0, 0)),
            pl.BlockSpec((1, HID), lambda t: (0, 0)),
        ],
        out_specs=pl.BlockSpec((_Q5, HID), lambda t: (t, 0)),
        out_shape=jax.ShapeDtypeStruct((Q, HID), jnp.float32),
    )(qp, rows, idxr, Wp, bp.reshape(1, HID))


def _build_tables(memory):
    n = jnp.linalg.norm(memory, axis=-1, keepdims=True)
    scale = jnp.minimum(1.0, MAXN / jnp.maximum(n, 1e-12))
    mp = memory * scale
    sm = jnp.sum(mp * mp, axis=-1, keepdims=True)
    table = jnp.concatenate(
        [memory, n, sm, jnp.zeros((M, TW - MD - 2), jnp.float32)], axis=1)
    pad_row = jnp.zeros((MPAD - M, TW), jnp.float32).at[:, MD + 1].set(jnp.inf)
    table = jnp.concatenate([table, pad_row], axis=0)
    npad = jnp.pad(n[:, 0], (0, MPAD - M))
    smpad = jnp.pad(sm[:, 0], (0, MPAD - M), constant_values=jnp.inf)
    aux = jnp.stack([npad.reshape(NCH, CHUNK), smpad.reshape(NCH, CHUNK)],
                    axis=1)
    return table, aux


def kernel(hidden, memory, W1, b1, ln_g, ln_b, W2, b2, q_origin, Wp, bp):
    table, aux = _build_tables(memory)
    qp = _qnet(hidden, W1, b1, ln_g, ln_b, W2, b2, q_origin)
    binmin = _scan(qp, table, aux)
    idxr = _select(binmin)
    rows = _gather_rows(table, idxr.reshape(-1))
    return _final(qp, rows.reshape(Q, CAND, TW), idxr, Wp, bp)


# speedup vs baseline: 5.2128x; 1.0324x over previous
import jax
import jax.numpy as jnp
from jax import lax
from jax.experimental import pallas as pl
from jax.experimental.pallas import tpu as pltpu
from jax.experimental.pallas import tpu_sc as plsc

HID = 1024
MD = 64
M = 100000
Q = 1024
K = 16
ALPHA = 0.1
MAXN = 1.0 - 1e-5

CHUNK = 2048
NCH = 49
MPAD = NCH * CHUNK
TW = 128
BIN = 16
BPC = CHUNK // BIN
NBIN = NCH * BPC
CAND = K * BIN

_Q3 = 128
_Q5 = 64


def _qnet_body(hidden_ref, w1_ref, b1_ref, g_ref, b_ref, w2_ref, b2_ref,
               qo_ref, qp_ref):
    x = jnp.dot(hidden_ref[...], w1_ref[...],
                preferred_element_type=jnp.float32) + b1_ref[...]
    mu = jnp.mean(x, axis=-1, keepdims=True)
    var = jnp.var(x, axis=-1, keepdims=True)
    x = (x - mu) / jnp.sqrt(var + 1e-5) * g_ref[...] + b_ref[...]
    x = jax.nn.gelu(x)
    q = jnp.dot(x, w2_ref[...],
                preferred_element_type=jnp.float32) + b2_ref[...] + qo_ref[...]
    n = jnp.sqrt(jnp.sum(q * q, axis=-1, keepdims=True))
    scale = jnp.minimum(1.0, MAXN / jnp.maximum(n, 1e-12))
    qp_ref[...] = q * scale


def _scan_body(qp_ref, t_ref, a_ref, binmin_ref):
    mem = t_ref[:, :MD]
    n = t_ref[:, MD:MD + 1]
    scale = jnp.minimum(1.0, MAXN / jnp.maximum(n, 1e-12))
    mp = mem * scale
    sm = a_ref[0, 1:2, :]
    invm = 1.0 / jnp.maximum(1.0 - sm, 1e-9)
    qp = qp_ref[...]
    sq = jnp.sum(qp * qp, axis=-1, keepdims=True)
    smc = t_ref[:, MD + 1:MD + 2]
    invc = 1.0 / jnp.maximum(1.0 - smc, 1e-9)
    mps = mp * (2.0 * invc)
    g3 = lax.dot_general(qp, mps, (((1,), (1,)), ((), ())),
                         preferred_element_type=jnp.float32)
    r = (sq + 1.0) * invm - g3
    binmin_ref[...] = jnp.min(r.reshape(Q, BIN, BPC), axis=1)


def _select_body(binmin_ref, idxr_ref):
    v = binmin_ref[...]
    bid = lax.broadcasted_iota(jnp.int32, (_Q3, NBIN), 1)
    tix = lax.broadcasted_iota(jnp.int32, (_Q3, BIN), 1)
    imax = jnp.int32(2**31 - 1)
    for j in range(K):
        m = jnp.min(v, axis=-1, keepdims=True)
        elig = v <= m
        sel = jnp.min(jnp.where(elig, bid, imax), axis=-1, keepdims=True)
        v = jnp.where(bid == sel, jnp.inf, v)
        base = (sel // BPC) * CHUNK + (sel % BPC)
        idxr_ref[:, j * BIN:(j + 1) * BIN] = base + BPC * tix


def _final_body(qp_ref, rows_ref, idxr_ref, wp_ref, bp_ref, out_ref):
    qp = qp_ref[...]
    raw = rows_ref[:, :, :MD]
    n = rows_ref[:, :, MD]
    sm = rows_ref[:, :, MD + 1]
    idx = idxr_ref[...]
    scale = jnp.minimum(1.0, MAXN / jnp.maximum(n, 1e-12))
    mp = raw * scale[..., None]
    sq = jnp.sum(qp * qp, axis=-1, keepdims=True)
    dot = jnp.sum(mp * qp[:, None, :], axis=-1)
    d2 = jnp.maximum(sq + sm - 2.0 * dot, 0.0)
    denom = jnp.maximum((1.0 - sq) * (1.0 - sm), 1e-12)
    arg = jnp.maximum(1.0 + 2.0 * d2 / denom, 1.0 + 1e-7)
    dist = jnp.log(arg + jnp.sqrt(arg * arg - 1.0))
    dist = jnp.where(idx < M, dist, jnp.inf)
    d0 = dist
    selmask = jnp.zeros(dist.shape, dtype=jnp.bool_)
    imax = jnp.int32(2**31 - 1)
    for j in range(K):
        m = jnp.min(dist, axis=-1, keepdims=True)
        elig = dist <= m
        sel = jnp.min(jnp.where(elig, idx, imax), axis=-1, keepdims=True)
        pick = elig & (idx == sel)
        selmask = selmask | pick
        dist = jnp.where(pick, jnp.inf, dist)
    dmin = jnp.min(d0, axis=-1, keepdims=True)
    w = jnp.where(selmask, jnp.exp((dmin - d0) / ALPHA), 0.0)
    w = w / jnp.sum(w, axis=-1, keepdims=True)
    retrieved = jnp.sum(raw * w[..., None], axis=1)
    out_ref[...] = jnp.dot(retrieved, wp_ref[...],
                           preferred_element_type=jnp.float32) + bp_ref[...]


_SC_B = Q * CAND
_NW = 32
_BPW = _SC_B // _NW
_SC_CH = 512


def _gather_body(tab_hbm, idx_hbm, out_hbm, idx_v, rows_v, sem):
    wid = lax.axis_index("s") * 2 + lax.axis_index("c")
    base = wid * _BPW

    def step(i, carry):
        off = base + i * _SC_CH
        pltpu.sync_copy(idx_hbm.at[pl.ds(off, _SC_CH)], idx_v)
        pltpu.async_copy(tab_hbm.at[idx_v], rows_v, sem).wait()
        pltpu.sync_copy(rows_v, out_hbm.at[pl.ds(off, _SC_CH)])
        return carry

    lax.fori_loop(0, _BPW // _SC_CH, step, 0)


def _gather_rows(table, idx_flat):
    mesh = plsc.VectorSubcoreMesh(core_axis_name="c", subcore_axis_name="s")
    run = pl.kernel(
        _gather_body,
        mesh=mesh,
        out_type=jax.ShapeDtypeStruct((_SC_B, TW), jnp.float32),
        scratch_types=[
            pltpu.VMEM((_SC_CH,), jnp.int32),
            pltpu.VMEM((_SC_CH, TW), jnp.float32),
            pltpu.SemaphoreType.DMA,
        ],
    )
    return run(table, idx_flat)


def _qnet(hidden, W1, b1, ln_g, ln_b, W2, b2, q_origin):
    return pl.pallas_call(
        _qnet_body,
        out_shape=jax.ShapeDtypeStruct((Q, MD), jnp.float32),
    )(hidden, W1, b1.reshape(1, MD), ln_g.reshape(1, MD), ln_b.reshape(1, MD),
      W2, b2.reshape(1, MD), q_origin)


def _scan(qp, table, aux):
    return pl.pallas_call(
        _scan_body,
        grid=(NCH,),
        in_specs=[
            pl.BlockSpec((Q, MD), lambda c: (0, 0)),
            pl.BlockSpec((CHUNK, TW), lambda c: (c, 0)),
            pl.BlockSpec((1, 2, CHUNK), lambda c: (c, 0, 0)),
        ],
        out_specs=pl.BlockSpec((Q, BPC), lambda c: (0, c)),
        out_shape=jax.ShapeDtypeStruct((Q, NBIN), jnp.float32),
    )(qp, table, aux)


def _select(binmin):
    return pl.pallas_call(
        _select_body,
        grid=(Q // _Q3,),
        in_specs=[pl.BlockSpec((_Q3, NBIN), lambda t: (t, 0))],
        out_specs=pl.BlockSpec((_Q3, CAND), lambda t: (t, 0)),
        out_shape=jax.ShapeDtypeStruct((Q, CAND), jnp.int32),
    )(binmin)


def _final(qp, rows, idxr, Wp, bp):
    return pl.pallas_call(
        _final_body,
        grid=(Q // _Q5,),
        in_specs=[
            pl.BlockSpec((_Q5, MD), lambda t: (t, 0)),
            pl.BlockSpec((_Q5, CAND, TW), lambda t: (t, 0, 0)),
            pl.BlockSpec((_Q5, CAND), lambda t: (t, 0)),
            pl.BlockSpec((MD, HID), lambda t: (0, 0)),
            pl.BlockSpec((1, HID), lambda t: (0, 0)),
        ],
        out_specs=pl.BlockSpec((_Q5, HID), lambda t: (t, 0)),
        out_shape=jax.ShapeDtypeStruct((Q, HID), jnp.float32),
    )(qp, rows, idxr, Wp, bp.reshape(1, HID))


def _build_tables(memory):
    n = jnp.linalg.norm(memory, axis=-1, keepdims=True)
    scale = jnp.minimum(1.0, MAXN / jnp.maximum(n, 1e-12))
    mp = memory * scale
    sm = jnp.sum(mp * mp, axis=-1, keepdims=True)
    table = jnp.concatenate(
        [memory, n, sm, jnp.zeros((M, TW - MD - 2), jnp.float32)], axis=1)
    pad_row = jnp.zeros((MPAD - M, TW), jnp.float32).at[:, MD + 1].set(jnp.inf)
    table = jnp.concatenate([table, pad_row], axis=0)
    npad = jnp.pad(n[:, 0], (0, MPAD - M))
    smpad = jnp.pad(sm[:, 0], (0, MPAD - M), constant_values=jnp.inf)
    aux = jnp.stack([npad.reshape(NCH, CHUNK), smpad.reshape(NCH, CHUNK)],
                    axis=1)
    return table, aux


def kernel(hidden, memory, W1, b1, ln_g, ln_b, W2, b2, q_origin, Wp, bp):
    table, aux = _build_tables(memory)
    qp = _qnet(hidden, W1, b1, ln_g, ln_b, W2, b2, q_origin)
    binmin = _scan(qp, table, aux)
    idxr = _select(binmin)
    rows = _gather_rows(table, idxr.reshape(-1))
    return _final(qp, rows.reshape(Q, CAND, TW), idxr, Wp, bp)
